# baseline (speedup 1.0000x reference)
import functools
import math

import jax
import jax.numpy as jnp
from jax import lax
from jax.experimental import pallas as pl
from jax.experimental.pallas import tpu as pltpu
from jax.experimental.pallas import tpu_sc as plsc

NC = 2
NS = 16
NW = NC * NS
C = 256
ZBLK = 128
PHASES = 2
LAMDA = 0.5


def _pad_rows(n):
    return -(-(n + 1) // (ZBLK * NS)) * (ZBLK * NS) - n


def _zero_rows(ref, nrows, ncols):
    z = jnp.zeros((16,), jnp.float32)

    def body(i, _):
        for k in range(ncols // 16):
            ref[i, pl.ds(k * 16, 16)] = z
        return 0

    lax.fori_loop(0, nrows, body, 0)


def _copy_out(acc, dst, n, s):
    base_rows = (n // (8 * NS)) * 8
    tail = n - base_rows * NS
    obase = s * base_rows
    pltpu.sync_copy(acc.at[pl.ds(obase, base_rows)],
                    dst.at[pl.ds(obase, base_rows)])
    if tail:
        @pl.when(s == 0)
        def _():
            pltpu.sync_copy(acc.at[pl.ds(base_rows * NS, tail)],
                            dst.at[pl.ds(base_rows * NS, tail)])


def _zero_acc(acc, zsrc, n_pad, s):
    blocks_per_tile = n_pad // (ZBLK * NS)
    for b in range(blocks_per_tile):
        pltpu.sync_copy(
            zsrc, acc.at[pl.ds((s * blocks_per_tile + b) * ZBLK, ZBLK)])


def _make_spmm_kernel(n, d, k_chunks):
    n_pad = n + _pad_rows(n)
    assert k_chunks % PHASES == 0
    kp = k_chunks // PHASES
    ew = kp * C

    mesh = plsc.VectorSubcoreMesh(core_axis_name="c", subcore_axis_name="s")

    @functools.partial(
        pl.kernel,
        out_type=jax.ShapeDtypeStruct((NC, n, d), jnp.float32),
        mesh=mesh,
        scratch_types=[
            pltpu.VMEM((ew,), jnp.int32),
            pltpu.VMEM((ew,), jnp.int32),
            pltpu.VMEM((C, d), jnp.float32),
            pltpu.VMEM_SHARED((n_pad, d), jnp.float32),
            pltpu.SemaphoreType.DMA,
            pltpu.SemaphoreType.DMA,
        ],
    )
    def spmm(g_hbm, col_hbm, row_hbm, p_hbm, col_v, row_v, rows_v, acc,
             gsem, ssem):
        c = lax.axis_index("c")
        s = lax.axis_index("s")

        _zero_rows(rows_v.at[pl.ds(0, ZBLK)], ZBLK, d)
        _zero_acc(acc, rows_v.at[pl.ds(0, ZBLK)], n_pad, s)

        w = c * NS + s
        pltpu.sync_copy(col_hbm.at[w, pl.ds(0, ew)], col_v)
        pltpu.sync_copy(row_hbm.at[w, pl.ds(0, ew)], row_v)
        plsc.subcore_barrier()

        for p in range(PHASES):
            if p > 0:
                pltpu.sync_copy(col_hbm.at[w, pl.ds(p * ew, ew)], col_v)
                pltpu.sync_copy(row_hbm.at[w, pl.ds(p * ew, ew)], row_v)

            def chunk(j, _):
                off = pl.multiple_of(j * C, C)
                pltpu.async_copy(g_hbm.at[col_v.at[pl.ds(off, C)]],
                                 rows_v, gsem).wait()
                pltpu.async_copy(rows_v, acc.at[row_v.at[pl.ds(off, C)]],
                                 ssem, add=True).wait()
                return 0

            lax.fori_loop(0, kp, chunk, 0)

        plsc.subcore_barrier()

        _copy_out(acc, p_hbm.at[c], n, s)

    return spmm


def _make_deg_kernel(n, k_chunks):
    n_pad = n + _pad_rows(n)
    w = 128
    ew = k_chunks * C

    mesh = plsc.VectorSubcoreMesh(core_axis_name="c", subcore_axis_name="s")

    @functools.partial(
        pl.kernel,
        out_type=jax.ShapeDtypeStruct((NC, n, w), jnp.float32),
        mesh=mesh,
        scratch_types=[
            pltpu.VMEM((ew,), jnp.int32),
            pltpu.VMEM((C, w), jnp.float32),
            pltpu.VMEM_SHARED((n_pad, w), jnp.float32),
            pltpu.SemaphoreType.DMA,
        ],
    )
    def deg(row_hbm, d_hbm, row_v, ones_v, acc, sem):
        c = lax.axis_index("c")
        s = lax.axis_index("s")

        _zero_rows(ones_v, C, w)
        _zero_acc(acc, ones_v.at[pl.ds(0, ZBLK)], n_pad, s)

        one = jnp.ones((16,), jnp.float32)

        def fill(i, _):
            for k in range(w // 16):
                ones_v[i, pl.ds(k * 16, 16)] = one
            return 0

        lax.fori_loop(0, C, fill, 0)

        w_id = c * NS + s
        pltpu.sync_copy(row_hbm.at[w_id], row_v)

        plsc.subcore_barrier()

        grp = 8 if k_chunks % 8 == 0 else 2

        def edge_body(t, _):
            for b in range(grp):
                off = pl.multiple_of((t * grp + b) * C, C)
                pltpu.async_copy(ones_v, acc.at[row_v.at[pl.ds(off, C)]],
                                 sem, add=True)
            for b in range(grp):
                off = pl.multiple_of((t * grp + b) * C, C)
                pltpu.make_async_copy(ones_v, acc.at[row_v.at[pl.ds(off, C)]],
                                      sem).wait()
            return 0

        lax.fori_loop(0, k_chunks // grp, edge_body, 0)

        plsc.subcore_barrier()

        _copy_out(acc, d_hbm.at[c], n, s)

    return deg


def _pre_tc(deg0_ref, deg1_ref, x_ref, dinv_ref, g_ref):
    deg = deg0_ref[...] + deg1_ref[...]
    dinv = jnp.where(deg == 0.0, 1.0, lax.rsqrt(jnp.maximum(deg, 1e-30)))
    dinv_ref[...] = dinv
    g_ref[...] = dinv * x_ref[...]


def _layer_tc(h_ref, x_ref, p_ref, dinv_ref, w_ref, qw_ref, qb_ref,
              h1_ref, g1_ref, *, theta):
    h = h_ref[...]
    dinv = dinv_ref[...]
    alpha = jax.nn.sigmoid(
        jnp.dot(h, qw_ref[...], preferred_element_type=jnp.float32)
        + qb_ref[0, 0] - 1.0)
    hi = dinv * (p_ref[0] + p_ref[1])
    support = (1.0 - alpha) * hi + alpha * x_ref[...]
    out = theta * jnp.dot(support, w_ref[...],
                          preferred_element_type=jnp.float32) \
        + (1.0 - theta) * support
    h1 = jnp.maximum(out, 0.0)
    h1_ref[...] = h1
    g1_ref[...] = dinv * h1


def _final_tc(h_ref, x_ref, p_ref, dinv_ref, w_ref, qw_ref, qb_ref,
              cw_ref, cb_ref, y_ref, *, theta):
    h = h_ref[...]
    dinv = dinv_ref[...]
    alpha = jax.nn.sigmoid(
        jnp.dot(h, qw_ref[...], preferred_element_type=jnp.float32)
        + qb_ref[0, 0] - 1.0)
    hi = dinv * (p_ref[0] + p_ref[1])
    support = (1.0 - alpha) * hi + alpha * x_ref[...]
    out = theta * jnp.dot(support, w_ref[...],
                          preferred_element_type=jnp.float32) \
        + (1.0 - theta) * support
    h1 = jnp.maximum(out, 0.0) + x_ref[...]
    y_ref[...] = jnp.dot(h1, cw_ref[...],
                         preferred_element_type=jnp.float32) + cb_ref[...]


def kernel(x, Ws, q_w, q_b, c_w, c_b, adj):
    n, d = x.shape
    e = adj.shape[1]
    layers = Ws.shape[0]
    out_dim = c_w.shape[1]

    k_chunks = -(-e // (NW * C))
    k_chunks = -(-k_chunks // (PHASES * 4)) * (PHASES * 4)
    e_pad = NW * C * k_chunks
    pad = e_pad - e
    row = adj[0]
    col = adj[1]
    if pad:
        npad_rows = _pad_rows(n)
        ar = jnp.arange(pad, dtype=jnp.int32)
        row = jnp.concatenate([row, n + (ar % npad_rows)])
        col = jnp.concatenate([col, (ar * 97) % n])
    row2 = row.reshape(NW, k_chunks * C)
    col2 = col.reshape(NW, k_chunks * C)

    spmm = _make_spmm_kernel(n, d, k_chunks)
    degs = _make_deg_kernel(n, k_chunks)(row2)
    deg0 = degs[0, :, 0:1]
    deg1 = degs[1, :, 0:1]

    bn = 2000
    grid = (n // bn,)
    row_spec = pl.BlockSpec((bn, d), lambda i: (i, 0))
    col1_spec = pl.BlockSpec((bn, 1), lambda i: (i, 0))
    p_spec = pl.BlockSpec((NC, bn, d), lambda i: (0, i, 0))
    full = lambda shape: pl.BlockSpec(shape, lambda i: tuple(0 for _ in shape))

    dinv, g = pl.pallas_call(
        _pre_tc,
        grid=grid,
        in_specs=[col1_spec, col1_spec, row_spec],
        out_specs=[col1_spec, row_spec],
        out_shape=[
            jax.ShapeDtypeStruct((n, 1), jnp.float32),
            jax.ShapeDtypeStruct((n, d), jnp.float32),
        ],
    )(deg0, deg1, x)

    qb2 = q_b.reshape(1, 1)
    cw_pad = jnp.zeros((d, 128), jnp.float32).at[:, :out_dim].set(c_w)
    cb_pad = jnp.zeros((1, 128), jnp.float32).at[0, :out_dim].set(c_b)

    h = x
    for i in range(layers):
        theta = math.log(LAMDA / (i + 1) + 1.0)
        p = spmm(g, col2, row2)
        if i < layers - 1:
            h, g = pl.pallas_call(
                functools.partial(_layer_tc, theta=theta),
                grid=grid,
                in_specs=[row_spec, row_spec, p_spec, col1_spec,
                          full((d, d)), full((d, 1)), full((1, 1))],
                out_specs=[row_spec, row_spec],
                out_shape=[
                    jax.ShapeDtypeStruct((n, d), jnp.float32),
                    jax.ShapeDtypeStruct((n, d), jnp.float32),
                ],
            )(h, x, p, dinv, Ws[i], q_w, qb2)
        else:
            y = pl.pallas_call(
                functools.partial(_final_tc, theta=theta),
                grid=grid,
                in_specs=[row_spec, row_spec, p_spec, col1_spec,
                          full((d, d)), full((d, 1)), full((1, 1)),
                          full((d, 128)), full((1, 128))],
                out_specs=row_spec,
                out_shape=jax.ShapeDtypeStruct((n, 128), jnp.float32),
            )(h, x, p, dinv, Ws[i], q_w, qb2, cw_pad, cb_pad)

    return y[:, :out_dim]


# speedup vs baseline: 1.1234x; 1.1234x over previous
import functools
import math

import jax
import jax.numpy as jnp
from jax import lax
from jax.experimental import pallas as pl
from jax.experimental.pallas import tpu as pltpu
from jax.experimental.pallas import tpu_sc as plsc

NC = 2
NS = 16
NW = NC * NS
C = 256
ZBLK = 128
PHASES = 2
NBUF = 2
LAMDA = 0.5


def _pad_rows(n):
    return -(-(n + 1) // (ZBLK * NS)) * (ZBLK * NS) - n


def _zero_rows(ref, nrows, ncols):
    z = jnp.zeros((16,), jnp.float32)

    def body(i, _):
        for k in range(ncols // 16):
            ref[i, pl.ds(k * 16, 16)] = z
        return 0

    lax.fori_loop(0, nrows, body, 0)


def _copy_out(acc, dst, n, s):
    base_rows = (n // (8 * NS)) * 8
    tail = n - base_rows * NS
    obase = s * base_rows
    pltpu.sync_copy(acc.at[pl.ds(obase, base_rows)],
                    dst.at[pl.ds(obase, base_rows)])
    if tail:
        @pl.when(s == 0)
        def _():
            pltpu.sync_copy(acc.at[pl.ds(base_rows * NS, tail)],
                            dst.at[pl.ds(base_rows * NS, tail)])


def _zero_acc(acc, zsrc, n_pad, s):
    blocks_per_tile = n_pad // (ZBLK * NS)
    for b in range(blocks_per_tile):
        pltpu.sync_copy(
            zsrc, acc.at[pl.ds((s * blocks_per_tile + b) * ZBLK, ZBLK)])


def _make_spmm_kernel(n, d, e_per_tile):
    n_pad = n + _pad_rows(n)
    cs = 128
    nbuf = NBUF
    assert e_per_tile % (PHASES * cs * nbuf) == 0
    ew = e_per_tile // PHASES
    kp = ew // cs

    mesh = plsc.VectorSubcoreMesh(core_axis_name="c", subcore_axis_name="s")

    @functools.partial(
        pl.kernel,
        out_type=jax.ShapeDtypeStruct((NC, n, d), jnp.float32),
        mesh=mesh,
        scratch_types=[
            pltpu.VMEM((ew,), jnp.int32),
            pltpu.VMEM((ew,), jnp.int32),
            pltpu.VMEM((nbuf, cs, d), jnp.float32),
            pltpu.VMEM_SHARED((n_pad, d), jnp.float32),
            pltpu.SemaphoreType.DMA((nbuf,)),
            pltpu.SemaphoreType.DMA((nbuf,)),
        ],
    )
    def spmm(g_hbm, col_hbm, row_hbm, p_hbm, col_v, row_v, rows_v, acc,
             gsem, ssem):
        c = lax.axis_index("c")
        s = lax.axis_index("s")

        _zero_rows(rows_v.at[0], cs, d)
        _zero_acc(acc, rows_v.at[0, pl.ds(0, ZBLK)], n_pad, s)

        w = c * NS + s

        def gather(j, b):
            off = pl.multiple_of(j * cs, cs)
            return pltpu.async_copy(g_hbm.at[col_v.at[pl.ds(off, cs)]],
                                    rows_v.at[b], gsem.at[b])

        def gather_wait(j, b):
            off = pl.multiple_of(j * cs, cs)
            pltpu.make_async_copy(g_hbm.at[col_v.at[pl.ds(off, cs)]],
                                  rows_v.at[b], gsem.at[b]).wait()

        def scatter(j, b):
            off = pl.multiple_of(j * cs, cs)
            return pltpu.async_copy(rows_v.at[b],
                                    acc.at[row_v.at[pl.ds(off, cs)]],
                                    ssem.at[b], add=True)

        def scatter_wait(j, b):
            off = pl.multiple_of(j * cs, cs)
            pltpu.make_async_copy(rows_v.at[b],
                                  acc.at[row_v.at[pl.ds(off, cs)]],
                                  ssem.at[b]).wait()

        pltpu.sync_copy(col_hbm.at[w, pl.ds(0, ew)], col_v)
        pltpu.sync_copy(row_hbm.at[w, pl.ds(0, ew)], row_v)
        plsc.subcore_barrier()

        for p in range(PHASES):
            if p > 0:
                pltpu.sync_copy(col_hbm.at[w, pl.ds(p * ew, ew)], col_v)
                pltpu.sync_copy(row_hbm.at[w, pl.ds(p * ew, ew)], row_v)

            gather(0, 0)

            def outer(t, _):
                for b in range(nbuf):
                    j = t * nbuf + b
                    bb = (b + 1) % nbuf
                    gather_wait(j, b)
                    jj = j + 1 - nbuf

                    @pl.when(jj >= 0)
                    def _():
                        scatter_wait(jj, bb)

                    @pl.when(j + 1 < kp)
                    def _():
                        gather(j + 1, bb)

                    scatter(j, b)
                return 0

            lax.fori_loop(0, kp // nbuf, outer, 0)

            for t in range(nbuf - 1):
                j = kp - nbuf + 1 + t
                scatter_wait(j, j % nbuf)

        plsc.subcore_barrier()

        _copy_out(acc, p_hbm.at[c], n, s)

    return spmm


def _make_deg_kernel(n, e_per_tile):
    n_pad = n + _pad_rows(n)
    w = 128
    k_chunks = e_per_tile // C
    ew = e_per_tile

    mesh = plsc.VectorSubcoreMesh(core_axis_name="c", subcore_axis_name="s")

    @functools.partial(
        pl.kernel,
        out_type=jax.ShapeDtypeStruct((NC, n, w), jnp.float32),
        mesh=mesh,
        scratch_types=[
            pltpu.VMEM((ew,), jnp.int32),
            pltpu.VMEM((C, w), jnp.float32),
            pltpu.VMEM_SHARED((n_pad, w), jnp.float32),
            pltpu.SemaphoreType.DMA,
        ],
    )
    def deg(row_hbm, d_hbm, row_v, ones_v, acc, sem):
        c = lax.axis_index("c")
        s = lax.axis_index("s")

        _zero_rows(ones_v, C, w)
        _zero_acc(acc, ones_v.at[pl.ds(0, ZBLK)], n_pad, s)

        one = jnp.ones((16,), jnp.float32)

        def fill(i, _):
            for k in range(w // 16):
                ones_v[i, pl.ds(k * 16, 16)] = one
            return 0

        lax.fori_loop(0, C, fill, 0)

        w_id = c * NS + s
        pltpu.sync_copy(row_hbm.at[w_id], row_v)

        plsc.subcore_barrier()

        grp = 8 if k_chunks % 8 == 0 else 2

        def edge_body(t, _):
            for b in range(grp):
                off = pl.multiple_of((t * grp + b) * C, C)
                pltpu.async_copy(ones_v, acc.at[row_v.at[pl.ds(off, C)]],
                                 sem, add=True)
            for b in range(grp):
                off = pl.multiple_of((t * grp + b) * C, C)
                pltpu.make_async_copy(ones_v, acc.at[row_v.at[pl.ds(off, C)]],
                                      sem).wait()
            return 0

        lax.fori_loop(0, k_chunks // grp, edge_body, 0)

        plsc.subcore_barrier()

        _copy_out(acc, d_hbm.at[c], n, s)

    return deg


def _pre_tc(deg0_ref, deg1_ref, x_ref, dinv_ref, g_ref):
    deg = deg0_ref[...] + deg1_ref[...]
    dinv = jnp.where(deg == 0.0, 1.0, lax.rsqrt(jnp.maximum(deg, 1e-30)))
    dinv_ref[...] = dinv
    g_ref[...] = dinv * x_ref[...]


def _layer_tc(h_ref, x_ref, p_ref, dinv_ref, w_ref, qw_ref, qb_ref,
              h1_ref, g1_ref, *, theta):
    h = h_ref[...]
    dinv = dinv_ref[...]
    alpha = jax.nn.sigmoid(
        jnp.dot(h, qw_ref[...], preferred_element_type=jnp.float32)
        + qb_ref[0, 0] - 1.0)
    hi = dinv * (p_ref[0] + p_ref[1])
    support = (1.0 - alpha) * hi + alpha * x_ref[...]
    out = theta * jnp.dot(support, w_ref[...],
                          preferred_element_type=jnp.float32) \
        + (1.0 - theta) * support
    h1 = jnp.maximum(out, 0.0)
    h1_ref[...] = h1
    g1_ref[...] = dinv * h1


def _final_tc(h_ref, x_ref, p_ref, dinv_ref, w_ref, qw_ref, qb_ref,
              cw_ref, cb_ref, y_ref, *, theta):
    h = h_ref[...]
    dinv = dinv_ref[...]
    alpha = jax.nn.sigmoid(
        jnp.dot(h, qw_ref[...], preferred_element_type=jnp.float32)
        + qb_ref[0, 0] - 1.0)
    hi = dinv * (p_ref[0] + p_ref[1])
    support = (1.0 - alpha) * hi + alpha * x_ref[...]
    out = theta * jnp.dot(support, w_ref[...],
                          preferred_element_type=jnp.float32) \
        + (1.0 - theta) * support
    h1 = jnp.maximum(out, 0.0) + x_ref[...]
    y_ref[...] = jnp.dot(h1, cw_ref[...],
                         preferred_element_type=jnp.float32) + cb_ref[...]


def kernel(x, Ws, q_w, q_b, c_w, c_b, adj):
    n, d = x.shape
    e = adj.shape[1]
    layers = Ws.shape[0]
    out_dim = c_w.shape[1]

    m = max(C, PHASES * 128 * NBUF)
    e_per_tile = -(-e // (NW * m)) * m
    e_pad = NW * e_per_tile
    pad = e_pad - e
    row = adj[0]
    col = adj[1]
    if pad:
        npad_rows = _pad_rows(n)
        ar = jnp.arange(pad, dtype=jnp.int32)
        row = jnp.concatenate([row, n + (ar % npad_rows)])
        col = jnp.concatenate([col, (ar * 97) % n])
    row2 = row.reshape(NW, e_per_tile)
    col2 = col.reshape(NW, e_per_tile)

    spmm = _make_spmm_kernel(n, d, e_per_tile)
    degs = _make_deg_kernel(n, e_per_tile)(row2)
    deg0 = degs[0, :, 0:1]
    deg1 = degs[1, :, 0:1]

    bn = 2000
    grid = (n // bn,)
    row_spec = pl.BlockSpec((bn, d), lambda i: (i, 0))
    col1_spec = pl.BlockSpec((bn, 1), lambda i: (i, 0))
    p_spec = pl.BlockSpec((NC, bn, d), lambda i: (0, i, 0))
    full = lambda shape: pl.BlockSpec(shape, lambda i: tuple(0 for _ in shape))

    dinv, g = pl.pallas_call(
        _pre_tc,
        grid=grid,
        in_specs=[col1_spec, col1_spec, row_spec],
        out_specs=[col1_spec, row_spec],
        out_shape=[
            jax.ShapeDtypeStruct((n, 1), jnp.float32),
            jax.ShapeDtypeStruct((n, d), jnp.float32),
        ],
    )(deg0, deg1, x)

    qb2 = q_b.reshape(1, 1)
    cw_pad = jnp.zeros((d, 128), jnp.float32).at[:, :out_dim].set(c_w)
    cb_pad = jnp.zeros((1, 128), jnp.float32).at[0, :out_dim].set(c_b)

    h = x
    for i in range(layers):
        theta = math.log(LAMDA / (i + 1) + 1.0)
        p = spmm(g, col2, row2)
        if i < layers - 1:
            h, g = pl.pallas_call(
                functools.partial(_layer_tc, theta=theta),
                grid=grid,
                in_specs=[row_spec, row_spec, p_spec, col1_spec,
                          full((d, d)), full((d, 1)), full((1, 1))],
                out_specs=[row_spec, row_spec],
                out_shape=[
                    jax.ShapeDtypeStruct((n, d), jnp.float32),
                    jax.ShapeDtypeStruct((n, d), jnp.float32),
                ],
            )(h, x, p, dinv, Ws[i], q_w, qb2)
        else:
            y = pl.pallas_call(
                functools.partial(_final_tc, theta=theta),
                grid=grid,
                in_specs=[row_spec, row_spec, p_spec, col1_spec,
                          full((d, d)), full((d, 1)), full((1, 1)),
                          full((d, 128)), full((1, 128))],
                out_specs=row_spec,
                out_shape=jax.ShapeDtypeStruct((n, 128), jnp.float32),
            )(h, x, p, dinv, Ws[i], q_w, qb2, cw_pad, cb_pad)

    return y[:, :out_dim]
